# baseline (speedup 1.0000x reference)
import jax
import jax.numpy as jnp
from jax import lax
from jax.experimental import pallas as pl
from jax.experimental.pallas import tpu as pltpu
from jax.experimental.pallas import tpu_sc as plsc

N = 10000
D = 128
E = 320000
NC = 2
NS = 16
NW = NC * NS
CHUNK = 128
NKP = 2560
NK = NKP // NW
IB = 8
NBLK = NK // IB
ACC_ROWS = N + 8
RST = 624
RLEFT = N - NS * RST
DEG_W = 128

_MESH = plsc.VectorSubcoreMesh(core_axis_name="c", subcore_axis_name="s")


def _spmv_body(src_hbm, sidx_hbm, didx_hbm, zrows_hbm, out_hbm,
               acc, sidxA, didxA, sidxB, didxB, bufA, bufB,
               gA, gB, sA, sB):
    c = lax.axis_index("c")
    s = lax.axis_index("s")
    wid = s * NC + c
    r0 = s * RST
    pltpu.sync_copy(zrows_hbm.at[pl.ds(0, RST)], acc.at[pl.ds(r0, RST)])

    @pl.when(s == 0)
    def _():
        pltpu.sync_copy(zrows_hbm.at[pl.ds(0, RLEFT)],
                        acc.at[pl.ds(NS * RST, RLEFT)])

    plsc.subcore_barrier()
    base = wid * NK * CHUNK

    def pair(pp, carry):
        eA = base + (2 * pp) * CHUNK
        eB = eA + CHUNK
        pltpu.sync_copy(sidx_hbm.at[pl.ds(eA, CHUNK)], sidxA)
        pltpu.sync_copy(didx_hbm.at[pl.ds(eA, CHUNK)], didxA)
        pltpu.sync_copy(sidx_hbm.at[pl.ds(eB, CHUNK)], sidxB)
        pltpu.sync_copy(didx_hbm.at[pl.ds(eB, CHUNK)], didxB)
        dA = pltpu.async_copy(src_hbm.at[sidxA], bufA, gA)
        dB = pltpu.async_copy(src_hbm.at[sidxB], bufB, gB)
        dA.wait()
        scA = pltpu.async_copy(bufA, acc.at[didxA], sA, add=True)
        dB.wait()
        scB = pltpu.async_copy(bufB, acc.at[didxB], sB, add=True)
        scA.wait()
        scB.wait()
        return carry

    lax.fori_loop(0, NK // 2, pair, 0)
    plsc.subcore_barrier()
    pltpu.sync_copy(acc.at[pl.ds(r0, RST)], out_hbm.at[c, pl.ds(r0, RST)])

    @pl.when(s == 0)
    def _():
        pltpu.sync_copy(acc.at[pl.ds(NS * RST, RLEFT)],
                        out_hbm.at[c, pl.ds(NS * RST, RLEFT)])


_spmv = pl.kernel(
    _spmv_body,
    out_type=jax.ShapeDtypeStruct((NC, N, D), jnp.float32),
    mesh=_MESH,
    scratch_types=[
        pltpu.VMEM_SHARED((ACC_ROWS, D), jnp.float32),
        pltpu.VMEM((CHUNK,), jnp.int32),
        pltpu.VMEM((CHUNK,), jnp.int32),
        pltpu.VMEM((CHUNK,), jnp.int32),
        pltpu.VMEM((CHUNK,), jnp.int32),
        pltpu.VMEM((CHUNK, D), jnp.float32),
        pltpu.VMEM((CHUNK, D), jnp.float32),
        pltpu.SemaphoreType.DMA,
        pltpu.SemaphoreType.DMA,
        pltpu.SemaphoreType.DMA,
        pltpu.SemaphoreType.DMA,
    ],
)


def _deg_body(uidx_hbm, iidx_hbm, ones_hbm, zrows_hbm, out_hbm,
              acc, uidx_v, iidx_v, ones_v, dsem):
    c = lax.axis_index("c")
    s = lax.axis_index("s")
    wid = s * NC + c
    r0 = s * RST
    pltpu.sync_copy(uidx_hbm.at[pl.ds(wid * NK, NK)], uidx_v)
    pltpu.sync_copy(iidx_hbm.at[pl.ds(wid * NK, NK)], iidx_v)
    pltpu.sync_copy(ones_hbm, ones_v)

    for phase, idx_v in enumerate((uidx_v, iidx_v)):
        pltpu.sync_copy(zrows_hbm.at[pl.ds(0, RST)], acc.at[pl.ds(r0, RST)])

        @pl.when(s == 0)
        def _():
            pltpu.sync_copy(zrows_hbm.at[pl.ds(0, RLEFT)],
                            acc.at[pl.ds(NS * RST, RLEFT)])

        plsc.subcore_barrier()

        def fire(k, carry):
            pltpu.async_copy(ones_v, acc.at[idx_v.at[k]], dsem, add=True)
            return carry

        lax.fori_loop(0, NK, fire, 0)

        def drain(k, carry):
            pltpu.make_async_copy(ones_v, acc.at[idx_v.at[0]], dsem).wait()
            return carry

        lax.fori_loop(0, NK, drain, 0)
        plsc.subcore_barrier()
        pltpu.sync_copy(acc.at[pl.ds(r0, RST)],
                        out_hbm.at[c, phase, pl.ds(r0, RST)])

        @pl.when(s == 0)
        def _():
            pltpu.sync_copy(acc.at[pl.ds(NS * RST, RLEFT)],
                            out_hbm.at[c, phase, pl.ds(NS * RST, RLEFT)])


_deg = pl.kernel(
    _deg_body,
    out_type=jax.ShapeDtypeStruct((NC, 2, N, DEG_W), jnp.float32),
    mesh=_MESH,
    scratch_types=[
        pltpu.VMEM_SHARED((ACC_ROWS, DEG_W), jnp.float32),
        pltpu.VMEM((NK, CHUNK), jnp.int32),
        pltpu.VMEM((NK, CHUNK), jnp.int32),
        pltpu.VMEM((CHUNK, DEG_W), jnp.float32),
        pltpu.SemaphoreType.DMA,
    ],
)


def _combine_body(p_ref, d_ref, o_ref):
    ssum = p_ref[0] + p_ref[1]
    deg = d_ref[0, :, :1] + d_ref[1, :, :1]
    o_ref[...] = ssum / jnp.maximum(deg, 1.0)


_BR = 1000


def _combine(p, dpair):
    return pl.pallas_call(
        _combine_body,
        out_shape=jax.ShapeDtypeStruct((N, D), jnp.float32),
        grid=(N // _BR,),
        in_specs=[
            pl.BlockSpec((NC, _BR, D), lambda j: (0, j, 0)),
            pl.BlockSpec((NC, _BR, DEG_W), lambda j: (0, j, 0)),
        ],
        out_specs=pl.BlockSpec((_BR, D), lambda j: (j, 0)),
    )(p, dpair)


def kernel(user_emb, item_emb, edge_index):
    u = edge_index[0].astype(jnp.int32)
    i = edge_index[1].astype(jnp.int32)
    npad = NKP * CHUNK - E
    pad0 = jnp.zeros((npad,), jnp.int32)
    padd = jnp.full((npad,), N, jnp.int32)
    srcU = jnp.concatenate([u, pad0])
    srcI = jnp.concatenate([i, pad0])
    dstU = jnp.concatenate([u, padd])
    dstI = jnp.concatenate([i, padd])
    zrows = jnp.zeros((RST, D), jnp.float32)
    ones = jnp.ones((CHUNK, DEG_W), jnp.float32)
    degs = _deg(dstU.reshape(NKP, CHUNK), dstI.reshape(NKP, CHUNK),
                ones, zrows)
    du = degs[:, 0]
    di = degs[:, 1]
    h_u, h_i = user_emb, item_emb
    for _ in range(2):
        rst = _combine(_spmv(h_u, srcU, dstI, zrows), di)
        nu = _combine(_spmv(rst, srcI, dstU, zrows), du)
        rs = _combine(_spmv(h_i, srcI, dstU, zrows), du)
        ni = _combine(_spmv(rs, srcU, dstI, zrows), di)
        h_u, h_i = nu, ni
    return jnp.stack([h_u, h_i], axis=0)


# speedup vs baseline: 1.0754x; 1.0754x over previous
import jax
import jax.numpy as jnp
from jax import lax
from jax.experimental import pallas as pl
from jax.experimental.pallas import tpu as pltpu
from jax.experimental.pallas import tpu_sc as plsc

N = 10000
D = 128
E = 320000
NC = 2
NS = 16
NW = NC * NS
CHUNK = 128
NKP = 2560
NK = NKP // NW
IB = 8
NBLK = NK // IB
ACC_ROWS = N + 8
RST = 624
RLEFT = N - NS * RST
DEG_W = 128

_MESH = plsc.VectorSubcoreMesh(core_axis_name="c", subcore_axis_name="s")


NQ = NK // 4


def _spmv_body(src_hbm, sidx_hbm, didx_hbm, zrows_hbm, out_hbm,
               acc, sidx0, didx0, sidx1, didx1, sidx2, didx2, sidx3, didx3,
               rows_v, is0, is1, g0):
    c = lax.axis_index("c")
    s = lax.axis_index("s")
    wid = s * NC + c
    r0 = s * RST
    pltpu.sync_copy(zrows_hbm.at[pl.ds(0, RST)], acc.at[pl.ds(r0, RST)])

    @pl.when(s == 0)
    def _():
        pltpu.sync_copy(zrows_hbm.at[pl.ds(0, RLEFT)],
                        acc.at[pl.ds(NS * RST, RLEFT)])

    plsc.subcore_barrier()
    base = wid * NK * CHUNK

    def fire_idx(e0, bufs, sem):
        pltpu.async_copy(sidx_hbm.at[pl.ds(e0, CHUNK)], bufs[0], sem)
        pltpu.async_copy(didx_hbm.at[pl.ds(e0, CHUNK)], bufs[1], sem)
        pltpu.async_copy(sidx_hbm.at[pl.ds(e0 + CHUNK, CHUNK)], bufs[2], sem)
        pltpu.async_copy(didx_hbm.at[pl.ds(e0 + CHUNK, CHUNK)], bufs[3], sem)

    def wait_idx(bufs, sem):
        for bf in bufs:
            pltpu.make_async_copy(sidx_hbm.at[pl.ds(0, CHUNK)], bf,
                                  sem).wait()

    def gs(si, di):
        pltpu.async_copy(src_hbm.at[si], rows_v, g0).wait()
        pltpu.sync_copy(rows_v, acc.at[di], add=True)

    setX = (sidx0, didx0, sidx1, didx1)
    setY = (sidx2, didx2, sidx3, didx3)
    fire_idx(base, setX, is0)

    def quad(qq, carry):
        b0 = base + (4 * qq) * CHUNK
        wait_idx(setX, is0)
        fire_idx(b0 + 2 * CHUNK, setY, is1)
        gs(sidx0, didx0)
        gs(sidx1, didx1)
        wait_idx(setY, is1)

        @pl.when(qq < NQ - 1)
        def _():
            fire_idx(b0 + 4 * CHUNK, setX, is0)

        gs(sidx2, didx2)
        gs(sidx3, didx3)
        return carry

    lax.fori_loop(0, NQ, quad, 0)
    plsc.subcore_barrier()
    pltpu.sync_copy(acc.at[pl.ds(r0, RST)], out_hbm.at[c, pl.ds(r0, RST)])

    @pl.when(s == 0)
    def _():
        pltpu.sync_copy(acc.at[pl.ds(NS * RST, RLEFT)],
                        out_hbm.at[c, pl.ds(NS * RST, RLEFT)])


_spmv = pl.kernel(
    _spmv_body,
    out_type=jax.ShapeDtypeStruct((NC, N, D), jnp.float32),
    mesh=_MESH,
    scratch_types=[
        pltpu.VMEM_SHARED((ACC_ROWS, D), jnp.float32),
        pltpu.VMEM((CHUNK,), jnp.int32),
        pltpu.VMEM((CHUNK,), jnp.int32),
        pltpu.VMEM((CHUNK,), jnp.int32),
        pltpu.VMEM((CHUNK,), jnp.int32),
        pltpu.VMEM((CHUNK,), jnp.int32),
        pltpu.VMEM((CHUNK,), jnp.int32),
        pltpu.VMEM((CHUNK,), jnp.int32),
        pltpu.VMEM((CHUNK,), jnp.int32),
        pltpu.VMEM((CHUNK, D), jnp.float32),
        pltpu.SemaphoreType.DMA,
        pltpu.SemaphoreType.DMA,
        pltpu.SemaphoreType.DMA,
    ],
)


def _deg_body(uidx_hbm, iidx_hbm, ones_hbm, zrows_hbm, out_hbm,
              acc, uidx_v, iidx_v, ones_v, dsem):
    c = lax.axis_index("c")
    s = lax.axis_index("s")
    wid = s * NC + c
    r0 = s * RST
    pltpu.sync_copy(uidx_hbm.at[pl.ds(wid * NK, NK)], uidx_v)
    pltpu.sync_copy(iidx_hbm.at[pl.ds(wid * NK, NK)], iidx_v)
    pltpu.sync_copy(ones_hbm, ones_v)

    for phase, idx_v in enumerate((uidx_v, iidx_v)):
        pltpu.sync_copy(zrows_hbm.at[pl.ds(0, RST)], acc.at[pl.ds(r0, RST)])

        @pl.when(s == 0)
        def _():
            pltpu.sync_copy(zrows_hbm.at[pl.ds(0, RLEFT)],
                            acc.at[pl.ds(NS * RST, RLEFT)])

        plsc.subcore_barrier()

        def fire(k, carry):
            pltpu.async_copy(ones_v, acc.at[idx_v.at[k]], dsem, add=True)
            return carry

        lax.fori_loop(0, NK, fire, 0)

        def drain(k, carry):
            pltpu.make_async_copy(ones_v, acc.at[idx_v.at[0]], dsem).wait()
            return carry

        lax.fori_loop(0, NK, drain, 0)
        plsc.subcore_barrier()
        pltpu.sync_copy(acc.at[pl.ds(r0, RST)],
                        out_hbm.at[c, phase, pl.ds(r0, RST)])

        @pl.when(s == 0)
        def _():
            pltpu.sync_copy(acc.at[pl.ds(NS * RST, RLEFT)],
                            out_hbm.at[c, phase, pl.ds(NS * RST, RLEFT)])


_deg = pl.kernel(
    _deg_body,
    out_type=jax.ShapeDtypeStruct((NC, 2, N, DEG_W), jnp.float32),
    mesh=_MESH,
    scratch_types=[
        pltpu.VMEM_SHARED((ACC_ROWS, DEG_W), jnp.float32),
        pltpu.VMEM((NK, CHUNK), jnp.int32),
        pltpu.VMEM((NK, CHUNK), jnp.int32),
        pltpu.VMEM((CHUNK, DEG_W), jnp.float32),
        pltpu.SemaphoreType.DMA,
    ],
)


def _combine_body(p_ref, d_ref, o_ref):
    ssum = p_ref[0] + p_ref[1]
    deg = d_ref[0, :, :1] + d_ref[1, :, :1]
    o_ref[...] = ssum / jnp.maximum(deg, 1.0)


_BR = 1000


def _combine(p, dpair):
    return pl.pallas_call(
        _combine_body,
        out_shape=jax.ShapeDtypeStruct((N, D), jnp.float32),
        grid=(N // _BR,),
        in_specs=[
            pl.BlockSpec((NC, _BR, D), lambda j: (0, j, 0)),
            pl.BlockSpec((NC, _BR, DEG_W), lambda j: (0, j, 0)),
        ],
        out_specs=pl.BlockSpec((_BR, D), lambda j: (j, 0)),
    )(p, dpair)


def kernel(user_emb, item_emb, edge_index):
    u = edge_index[0].astype(jnp.int32)
    i = edge_index[1].astype(jnp.int32)
    npad = NKP * CHUNK - E
    pad0 = jnp.zeros((npad,), jnp.int32)
    padd = jnp.full((npad,), N, jnp.int32)
    srcU = jnp.concatenate([u, pad0])
    srcI = jnp.concatenate([i, pad0])
    dstU = jnp.concatenate([u, padd])
    dstI = jnp.concatenate([i, padd])
    zrows = jnp.zeros((RST, D), jnp.float32)
    ones = jnp.ones((CHUNK, DEG_W), jnp.float32)
    degs = _deg(dstU.reshape(NKP, CHUNK), dstI.reshape(NKP, CHUNK),
                ones, zrows)
    du = degs[:, 0]
    di = degs[:, 1]
    h_u, h_i = user_emb, item_emb
    for _ in range(2):
        rst = _combine(_spmv(h_u, srcU, dstI, zrows), di)
        nu = _combine(_spmv(rst, srcI, dstU, zrows), du)
        rs = _combine(_spmv(h_i, srcI, dstU, zrows), du)
        ni = _combine(_spmv(rs, srcU, dstI, zrows), di)
        h_u, h_i = nu, ni
    return jnp.stack([h_u, h_i], axis=0)


# speedup vs baseline: 2.2524x; 2.0945x over previous
import jax
import jax.numpy as jnp
from jax import lax
from jax.experimental import pallas as pl
from jax.experimental.pallas import tpu as pltpu
from jax.experimental.pallas import tpu_sc as plsc

N = 10000
D = 128
E = 320000
NC = 2
NS = 16
NW = NC * NS
CHUNK = 128
NKP = 2560
NK = NKP // NW
IB = 8
NBLK = NK // IB
ACC_ROWS = N + 8
RST = 624
RLEFT = N - NS * RST
DEG_W = 128

_MESH = plsc.VectorSubcoreMesh(core_axis_name="c", subcore_axis_name="s")


NCH = E // CHUNK


def _spmv_body(src_hbm, sidx_hbm, didx_hbm, zrows_hbm, out_hbm,
               acc, sidx_v, didx_v, rows_v, gsem):
    c = lax.axis_index("c")
    s = lax.axis_index("s")
    wid = s * NC + c
    r0 = s * RST
    pltpu.sync_copy(zrows_hbm.at[pl.ds(0, RST)], acc.at[pl.ds(r0, RST)])

    @pl.when(s == 0)
    def _():
        pltpu.sync_copy(zrows_hbm.at[pl.ds(0, RLEFT)],
                        acc.at[pl.ds(NS * RST, RLEFT)])

    plsc.subcore_barrier()
    cs = (wid * NCH) // NW
    ce = ((wid + 1) * NCH) // NW

    def step(n, carry):
        base = n * CHUNK
        pltpu.sync_copy(sidx_hbm.at[pl.ds(base, CHUNK)], sidx_v)
        pltpu.sync_copy(didx_hbm.at[pl.ds(base, CHUNK)], didx_v)
        pltpu.async_copy(src_hbm.at[sidx_v], rows_v, gsem).wait()
        pltpu.sync_copy(rows_v, acc.at[didx_v], add=True)
        return carry

    lax.fori_loop(cs, ce, step, 0)
    plsc.subcore_barrier()
    pltpu.sync_copy(acc.at[pl.ds(r0, RST)], out_hbm.at[c, pl.ds(r0, RST)])

    @pl.when(s == 0)
    def _():
        pltpu.sync_copy(acc.at[pl.ds(NS * RST, RLEFT)],
                        out_hbm.at[c, pl.ds(NS * RST, RLEFT)])


_spmv = pl.kernel(
    _spmv_body,
    out_type=jax.ShapeDtypeStruct((NC, N, D), jnp.float32),
    mesh=_MESH,
    scratch_types=[
        pltpu.VMEM_SHARED((ACC_ROWS, D), jnp.float32),
        pltpu.VMEM((CHUNK,), jnp.int32),
        pltpu.VMEM((CHUNK,), jnp.int32),
        pltpu.VMEM((CHUNK, D), jnp.float32),
        pltpu.SemaphoreType.DMA,
    ],
)


NK2 = NKP // NS


def _deg_body(uidx_hbm, iidx_hbm, ones_hbm, zrows_hbm, out_hbm,
              acc, idx_v, ones_v, dsem):
    c = lax.axis_index("c")
    s = lax.axis_index("s")
    r0 = s * RST

    @pl.when(c == 0)
    def _():
        pltpu.sync_copy(uidx_hbm.at[pl.ds(s * NK2, NK2)], idx_v)

    @pl.when(c == 1)
    def _():
        pltpu.sync_copy(iidx_hbm.at[pl.ds(s * NK2, NK2)], idx_v)

    pltpu.sync_copy(ones_hbm, ones_v)
    pltpu.sync_copy(zrows_hbm.at[pl.ds(0, RST)], acc.at[pl.ds(r0, RST)])

    @pl.when(s == 0)
    def _():
        pltpu.sync_copy(zrows_hbm.at[pl.ds(0, RLEFT)],
                        acc.at[pl.ds(NS * RST, RLEFT)])

    plsc.subcore_barrier()

    def fire(k, carry):
        pltpu.async_copy(ones_v, acc.at[idx_v.at[k]], dsem, add=True)
        return carry

    lax.fori_loop(0, NK2, fire, 0)

    def drain(k, carry):
        pltpu.make_async_copy(ones_v, acc.at[idx_v.at[0]], dsem).wait()
        return carry

    lax.fori_loop(0, NK2, drain, 0)
    plsc.subcore_barrier()
    pltpu.sync_copy(acc.at[pl.ds(r0, RST)], out_hbm.at[c, pl.ds(r0, RST)])

    @pl.when(s == 0)
    def _():
        pltpu.sync_copy(acc.at[pl.ds(NS * RST, RLEFT)],
                        out_hbm.at[c, pl.ds(NS * RST, RLEFT)])


_deg = pl.kernel(
    _deg_body,
    out_type=jax.ShapeDtypeStruct((NC, N, DEG_W), jnp.float32),
    mesh=_MESH,
    scratch_types=[
        pltpu.VMEM_SHARED((ACC_ROWS, DEG_W), jnp.float32),
        pltpu.VMEM((NK2, CHUNK), jnp.int32),
        pltpu.VMEM((CHUNK, DEG_W), jnp.float32),
        pltpu.SemaphoreType.DMA,
    ],
)


def _combine_body(p_ref, d_ref, o_ref):
    ssum = p_ref[0] + p_ref[1]
    o_ref[...] = ssum / jnp.maximum(d_ref[:, :1], 1.0)


_BR = 1000


def _combine(p, deg):
    return pl.pallas_call(
        _combine_body,
        out_shape=jax.ShapeDtypeStruct((N, D), jnp.float32),
        grid=(N // _BR,),
        in_specs=[
            pl.BlockSpec((NC, _BR, D), lambda j: (0, j, 0)),
            pl.BlockSpec((_BR, DEG_W), lambda j: (j, 0)),
        ],
        out_specs=pl.BlockSpec((_BR, D), lambda j: (j, 0)),
    )(p, deg)


def kernel(user_emb, item_emb, edge_index):
    u = edge_index[0].astype(jnp.int32)
    i = edge_index[1].astype(jnp.int32)
    npad = NKP * CHUNK - E
    padd = jnp.full((npad,), N, jnp.int32)
    dstU = jnp.concatenate([u, padd]).reshape(NKP, CHUNK)
    dstI = jnp.concatenate([i, padd]).reshape(NKP, CHUNK)
    zrows = jnp.zeros((RST, D), jnp.float32)
    ones = jnp.ones((CHUNK, DEG_W), jnp.float32)
    degs = _deg(dstU, dstI, ones, zrows)
    du = degs[0]
    di = degs[1]
    h_u, h_i = user_emb, item_emb
    for _ in range(2):
        rst = _combine(_spmv(h_u, u, i, zrows), di)
        nu = _combine(_spmv(rst, i, u, zrows), du)
        rs = _combine(_spmv(h_i, i, u, zrows), du)
        ni = _combine(_spmv(rs, u, i, zrows), di)
        h_u, h_i = nu, ni
    return jnp.stack([h_u, h_i], axis=0)


# speedup vs baseline: 2.5722x; 1.1420x over previous
import jax
import jax.numpy as jnp
from jax import lax
from jax.experimental import pallas as pl
from jax.experimental.pallas import tpu as pltpu
from jax.experimental.pallas import tpu_sc as plsc

N = 10000
D = 128
E = 320000
NC = 2
NS = 16
NW = NC * NS
CHUNK = 128
NKP = 2560
NK = NKP // NW
IB = 8
NBLK = NK // IB
ACC_ROWS = N + 8
RST = 624
RLEFT = N - NS * RST
DEG_W = 128

_MESH = plsc.VectorSubcoreMesh(core_axis_name="c", subcore_axis_name="s")


NCH = E // CHUNK


def _spmv_body(src_hbm, sidx_hbm, didx_hbm, zrows_hbm, out_hbm,
               acc, sidx_v, didx_v, rows_v, gsem, isem):
    c = lax.axis_index("c")
    s = lax.axis_index("s")
    wid = s * NC + c
    r0 = s * RST
    pltpu.sync_copy(zrows_hbm.at[pl.ds(0, RST)], acc.at[pl.ds(r0, RST)])

    @pl.when(s == 0)
    def _():
        pltpu.sync_copy(zrows_hbm.at[pl.ds(0, RLEFT)],
                        acc.at[pl.ds(NS * RST, RLEFT)])

    plsc.subcore_barrier()
    cs = (wid * NCH) // NW
    ce = ((wid + 1) * NCH) // NW

    def step(n, carry):
        base = n * CHUNK
        da = pltpu.async_copy(sidx_hbm.at[pl.ds(base, CHUNK)], sidx_v, isem)
        db = pltpu.async_copy(didx_hbm.at[pl.ds(base, CHUNK)], didx_v, gsem)
        da.wait()
        db.wait()
        pltpu.async_copy(src_hbm.at[sidx_v], rows_v, gsem).wait()
        pltpu.sync_copy(rows_v, acc.at[didx_v], add=True)
        return carry

    lax.fori_loop(cs, ce, step, 0)
    plsc.subcore_barrier()
    pltpu.sync_copy(acc.at[pl.ds(r0, RST)], out_hbm.at[c, pl.ds(r0, RST)])

    @pl.when(s == 0)
    def _():
        pltpu.sync_copy(acc.at[pl.ds(NS * RST, RLEFT)],
                        out_hbm.at[c, pl.ds(NS * RST, RLEFT)])


_spmv = pl.kernel(
    _spmv_body,
    out_type=jax.ShapeDtypeStruct((NC, N, D), jnp.float32),
    mesh=_MESH,
    scratch_types=[
        pltpu.VMEM_SHARED((ACC_ROWS, D), jnp.float32),
        pltpu.VMEM((CHUNK,), jnp.int32),
        pltpu.VMEM((CHUNK,), jnp.int32),
        pltpu.VMEM((CHUNK, D), jnp.float32),
        pltpu.SemaphoreType.DMA,
        pltpu.SemaphoreType.DMA,
    ],
)


NK2 = NKP // NS


def _deg_body(uidx_hbm, iidx_hbm, ones_hbm, zrows_hbm, out_hbm,
              acc, idx_v, ones_v, dsem):
    c = lax.axis_index("c")
    s = lax.axis_index("s")
    r0 = s * RST

    @pl.when(c == 0)
    def _():
        pltpu.sync_copy(uidx_hbm.at[pl.ds(s * NK2, NK2)], idx_v)

    @pl.when(c == 1)
    def _():
        pltpu.sync_copy(iidx_hbm.at[pl.ds(s * NK2, NK2)], idx_v)

    pltpu.sync_copy(ones_hbm, ones_v)
    pltpu.sync_copy(zrows_hbm.at[pl.ds(0, RST)], acc.at[pl.ds(r0, RST)])

    @pl.when(s == 0)
    def _():
        pltpu.sync_copy(zrows_hbm.at[pl.ds(0, RLEFT)],
                        acc.at[pl.ds(NS * RST, RLEFT)])

    plsc.subcore_barrier()

    def fire(k, carry):
        pltpu.async_copy(ones_v, acc.at[idx_v.at[k]], dsem, add=True)
        return carry

    lax.fori_loop(0, NK2, fire, 0)

    def drain(k, carry):
        pltpu.make_async_copy(ones_v, acc.at[idx_v.at[0]], dsem).wait()
        return carry

    lax.fori_loop(0, NK2, drain, 0)
    plsc.subcore_barrier()
    pltpu.sync_copy(acc.at[pl.ds(r0, RST)], out_hbm.at[c, pl.ds(r0, RST)])

    @pl.when(s == 0)
    def _():
        pltpu.sync_copy(acc.at[pl.ds(NS * RST, RLEFT)],
                        out_hbm.at[c, pl.ds(NS * RST, RLEFT)])


_deg = pl.kernel(
    _deg_body,
    out_type=jax.ShapeDtypeStruct((NC, N, DEG_W), jnp.float32),
    mesh=_MESH,
    scratch_types=[
        pltpu.VMEM_SHARED((ACC_ROWS, DEG_W), jnp.float32),
        pltpu.VMEM((NK2, CHUNK), jnp.int32),
        pltpu.VMEM((CHUNK, DEG_W), jnp.float32),
        pltpu.SemaphoreType.DMA,
    ],
)


def _combine_body(p_ref, d_ref, o_ref):
    ssum = p_ref[0] + p_ref[1]
    o_ref[...] = ssum / jnp.maximum(d_ref[:, :1], 1.0)


_BR = 1000


def _combine(p, deg):
    return pl.pallas_call(
        _combine_body,
        out_shape=jax.ShapeDtypeStruct((N, D), jnp.float32),
        grid=(N // _BR,),
        in_specs=[
            pl.BlockSpec((NC, _BR, D), lambda j: (0, j, 0)),
            pl.BlockSpec((_BR, DEG_W), lambda j: (j, 0)),
        ],
        out_specs=pl.BlockSpec((_BR, D), lambda j: (j, 0)),
    )(p, deg)


def kernel(user_emb, item_emb, edge_index):
    u = edge_index[0].astype(jnp.int32)
    i = edge_index[1].astype(jnp.int32)
    npad = NKP * CHUNK - E
    padd = jnp.full((npad,), N, jnp.int32)
    dstU = jnp.concatenate([u, padd]).reshape(NKP, CHUNK)
    dstI = jnp.concatenate([i, padd]).reshape(NKP, CHUNK)
    zrows = jnp.zeros((RST, D), jnp.float32)
    ones = jnp.ones((CHUNK, DEG_W), jnp.float32)
    degs = _deg(dstU, dstI, ones, zrows)
    du = degs[0]
    di = degs[1]
    h_u, h_i = user_emb, item_emb
    for _ in range(2):
        rst = _combine(_spmv(h_u, u, i, zrows), di)
        nu = _combine(_spmv(rst, i, u, zrows), du)
        rs = _combine(_spmv(h_i, i, u, zrows), du)
        ni = _combine(_spmv(rs, u, i, zrows), di)
        h_u, h_i = nu, ni
    return jnp.stack([h_u, h_i], axis=0)
